# Initial kernel scaffold; baseline (speedup 1.0000x reference)
import jax
import jax.numpy as jnp
from jax.experimental import pallas as pl


def kernel(x, edge_index, y, train_mask, weight, bias):
    raise NotImplementedError("write your pallas kernel here")


# speedup vs baseline: 4.7090x; 4.7090x over previous
import functools

import jax
import jax.numpy as jnp
from jax import lax
from jax.experimental import pallas as pl
from jax.experimental.pallas import tpu as pltpu
from jax.experimental.pallas import tpu_sc as plsc

NC = 2
NS = 16
NW = NC * NS
G = 128
CH = 128
NCLS = 40
ALPHA = 1.0
BETA = 1.0e-1
NPOW = 10


@functools.partial(jax.jit, static_argnames=("n_rows", "nb"))
def _spmm_call(v, colp, rowp, zeros_blk, *, n_rows, nb):
    d = v.shape[1]
    nch = n_rows // CH
    mesh = plsc.VectorSubcoreMesh(core_axis_name="c", subcore_axis_name="s")

    @functools.partial(
        pl.kernel,
        out_type=jax.ShapeDtypeStruct((NC, n_rows, d), jnp.float32),
        mesh=mesh,
        scratch_types=[
            pltpu.VMEM((G,), jnp.int32),
            pltpu.VMEM((G,), jnp.int32),
            pltpu.VMEM((G, d), jnp.float32),
            pltpu.VMEM_SHARED((n_rows, d), jnp.float32),
            pltpu.SemaphoreType.DMA,
            pltpu.SemaphoreType.DMA,
        ],
    )
    def spmm(v_hbm, colp_hbm, rowp_hbm, z_hbm, acc_hbm,
             cidx, ridx, rows, acc_sh, sem, sem2):
        c = lax.axis_index("c")
        s = lax.axis_index("s")
        wid = s * NC + c

        @pl.when(s == 0)
        def _():
            pltpu.sync_copy(z_hbm, rows)
            cps = [pltpu.async_copy(rows, acc_sh.at[pl.ds(k * CH, CH)], sem)
                   for k in range(nch)]
            for cp in cps:
                cp.wait()

        plsc.subcore_barrier()

        def body(b, carry):
            base = (wid * nb + b) * G
            pltpu.sync_copy(colp_hbm.at[pl.ds(base, G)], cidx)
            pltpu.sync_copy(rowp_hbm.at[pl.ds(base, G)], ridx)
            pltpu.async_copy(v_hbm.at[cidx], rows, sem2).wait()
            pltpu.sync_copy(rows, acc_sh.at[ridx], add=True)
            return carry

        lax.fori_loop(0, nb, body, 0, unroll=False)
        plsc.subcore_barrier()

        @pl.when(s == 0)
        def _():
            for k in range(nch):
                pltpu.sync_copy(acc_sh.at[pl.ds(k * CH, CH)], rows)
                pltpu.sync_copy(rows, acc_hbm.at[c, pl.ds(k * CH, CH)])

    return spmm(v, colp, rowp, zeros_blk)


def _prep_call(x, deg2, y2, tm2):
    n, d = x.shape

    def body(x_ref, deg_ref, y_ref, tm_ref, xc_ref, rinv_ref, yraw_ref, ysc_ref):
        xv = x_ref[...]
        xc_ref[...] = xv - jnp.mean(xv, axis=0, keepdims=True)
        degv = deg_ref[0, :n, 0:1] + deg_ref[1, :n, 0:1]
        rinv_ref[...] = 1.0 / degv
        cls_ids = lax.broadcasted_iota(jnp.int32, (1, NCLS), 1)
        yv = jnp.where((y_ref[...] == cls_ids) & (tm_ref[...] > 0.5), 1.0, 0.0)
        yraw_ref[...] = yv
        cnt = jnp.sum(yv, axis=0, keepdims=True)
        ysc_ref[...] = yv / (cnt + 1e-8)

    return pl.pallas_call(
        body,
        out_shape=(
            jax.ShapeDtypeStruct((n, d), jnp.float32),
            jax.ShapeDtypeStruct((n, 1), jnp.float32),
            jax.ShapeDtypeStruct((n, NCLS), jnp.float32),
            jax.ShapeDtypeStruct((n, NCLS), jnp.float32),
        ),
    )(x, deg2, y2, tm2)


def _combine_call(acc, v, xc, rinv, yraw, ysc):
    n, d = v.shape
    ca = ALPHA / (1.0 + ALPHA)
    cx = 1.0 / (1.0 + ALPHA)

    def body(acc_ref, v_ref, xc_ref, rinv_ref, yraw_ref, ysc_ref, out_ref):
        vv = v_ref[...]
        cls = lax.dot_general(yraw_ref[...], vv, (((0,), (0,)), ((), ())),
                              preferred_element_type=jnp.float32)
        part2 = jnp.dot(ysc_ref[...], cls, preferred_element_type=jnp.float32)
        p1 = (acc_ref[0, :n, :] + acc_ref[1, :n, :]) * rinv_ref[...]
        out_ref[...] = (ca * (1.0 - BETA)) * p1 + (ca * BETA) * part2 + cx * xc_ref[...]

    return pl.pallas_call(
        body,
        out_shape=jax.ShapeDtypeStruct((n, d), jnp.float32),
    )(acc, v, xc, rinv, yraw, ysc)


def _final_call(v, weight, bias):
    n = v.shape[0]
    nout = weight.shape[1]

    def body(v_ref, w_ref, b_ref, out_ref):
        out_ref[...] = (jnp.dot(v_ref[...], w_ref[...],
                                preferred_element_type=jnp.float32)
                        + b_ref[...])

    return pl.pallas_call(
        body,
        out_shape=jax.ShapeDtypeStruct((n, nout), jnp.float32),
    )(v, weight, bias)


def kernel(x, edge_index, y, train_mask, weight, bias):
    n, d = x.shape
    e = edge_index.shape[1]
    et = e + n

    n_rows = -(-(n + 8) // CH) * CH
    per_tile = -(-et // NW)
    per_tile = -(-per_tile // G) * G
    nb = per_tile // G
    tot = per_tile * NW
    pad = tot - et

    sl = jnp.arange(n, dtype=jnp.int32)
    row = jnp.concatenate([edge_index[0], sl,
                           n + (jnp.arange(pad, dtype=jnp.int32) % 8)])
    col = jnp.concatenate([edge_index[1], sl,
                           jnp.zeros((pad,), dtype=jnp.int32)])

    zeros_blk = jnp.zeros((CH, d), dtype=jnp.float32)
    ones_tbl = jnp.ones((n, d), dtype=jnp.float32)

    deg2 = _spmm_call(ones_tbl, col, row, zeros_blk, n_rows=n_rows, nb=nb)
    xc, rinv, yraw, ysc = _prep_call(
        x, deg2, y.reshape(n, 1),
        train_mask.reshape(n, 1).astype(jnp.float32))

    v = xc
    for _ in range(NPOW):
        acc = _spmm_call(v, col, row, zeros_blk, n_rows=n_rows, nb=nb)
        v = _combine_call(acc, v, xc, rinv, yraw, ysc)

    return _final_call(v, weight, bias)


# speedup vs baseline: 8.1630x; 1.7335x over previous
import functools

import jax
import jax.numpy as jnp
from jax import lax
from jax.experimental import pallas as pl
from jax.experimental.pallas import tpu as pltpu
from jax.experimental.pallas import tpu_sc as plsc

NC = 2
NS = 16
NW = NC * NS
G = 128
CH = 128
NCLS = 40
ALPHA = 1.0
BETA = 1.0e-1
NPOW = 10


@functools.partial(jax.jit, static_argnames=("n_rows", "nb"))
def _spmm_call(v, colp, rowp, zeros_blk, iota, *, n_rows, nb):
    d = v.shape[1]
    nch = n_rows // CH
    mesh = plsc.VectorSubcoreMesh(core_axis_name="c", subcore_axis_name="s")

    @functools.partial(
        pl.kernel,
        out_type=jax.ShapeDtypeStruct((NC, n_rows, d), jnp.float32),
        mesh=mesh,
        scratch_types=[
            pltpu.VMEM((G,), jnp.int32),
            pltpu.VMEM((G,), jnp.int32),
            pltpu.VMEM((G, d), jnp.float32),
            pltpu.VMEM_SHARED((n_rows, d), jnp.float32),
            pltpu.SemaphoreType.DMA,
            pltpu.SemaphoreType.DMA,
        ],
    )
    def spmm(v_hbm, colp_hbm, rowp_hbm, z_hbm, iota_hbm, acc_hbm,
             cidx, ridx, rows, acc_sh, sem, sem2):
        c = lax.axis_index("c")
        s = lax.axis_index("s")
        wid = s * NC + c

        rpt = n_rows // NS
        nck = rpt // CH
        pltpu.sync_copy(z_hbm, rows)
        for k in range(nck):
            pltpu.sync_copy(iota_hbm.at[pl.ds(s * rpt + k * CH, CH)], ridx)
            pltpu.async_copy(rows, acc_sh.at[ridx], sem).wait()

        plsc.subcore_barrier()

        def body(b, carry):
            base = (wid * nb + b) * G
            pltpu.sync_copy(colp_hbm.at[pl.ds(base, G)], cidx)
            pltpu.sync_copy(rowp_hbm.at[pl.ds(base, G)], ridx)
            pltpu.async_copy(v_hbm.at[cidx], rows, sem2).wait()
            pltpu.sync_copy(rows, acc_sh.at[ridx], add=True)
            return carry

        lax.fori_loop(0, nb, body, 0, unroll=False)
        plsc.subcore_barrier()

        for k in range(nck):
            pltpu.sync_copy(iota_hbm.at[pl.ds(s * rpt + k * CH, CH)], ridx)
            pltpu.async_copy(acc_sh.at[ridx], rows, sem2).wait()
            pltpu.sync_copy(rows, acc_hbm.at[c, pl.ds(s * rpt + k * CH, CH)])

    return spmm(v, colp, rowp, zeros_blk, iota)


def _prep_call(x, deg2, y2, tm2):
    n, d = x.shape

    def body(x_ref, deg_ref, y_ref, tm_ref, xc_ref, rinv_ref, yraw_ref, ysc_ref):
        xv = x_ref[...]
        xc_ref[...] = xv - jnp.mean(xv, axis=0, keepdims=True)
        degv = deg_ref[0, :n, 0:1] + deg_ref[1, :n, 0:1]
        rinv_ref[...] = 1.0 / degv
        cls_ids = lax.broadcasted_iota(jnp.int32, (1, NCLS), 1)
        yv = jnp.where((y_ref[...] == cls_ids) & (tm_ref[...] > 0.5), 1.0, 0.0)
        yraw_ref[...] = yv
        cnt = jnp.sum(yv, axis=0, keepdims=True)
        ysc_ref[...] = yv / (cnt + 1e-8)

    return pl.pallas_call(
        body,
        out_shape=(
            jax.ShapeDtypeStruct((n, d), jnp.float32),
            jax.ShapeDtypeStruct((n, 1), jnp.float32),
            jax.ShapeDtypeStruct((n, NCLS), jnp.float32),
            jax.ShapeDtypeStruct((n, NCLS), jnp.float32),
        ),
    )(x, deg2, y2, tm2)


def _rest_call(v, xc, yraw, ysc):
    n, d = v.shape
    ca = ALPHA / (1.0 + ALPHA)
    cx = 1.0 / (1.0 + ALPHA)

    def body(v_ref, xc_ref, yraw_ref, ysc_ref, out_ref):
        vv = v_ref[...]
        cls = lax.dot_general(yraw_ref[...], vv, (((0,), (0,)), ((), ())),
                              preferred_element_type=jnp.float32)
        part2 = jnp.dot(ysc_ref[...], cls, preferred_element_type=jnp.float32)
        out_ref[...] = (ca * BETA) * part2 + cx * xc_ref[...]

    return pl.pallas_call(
        body,
        out_shape=jax.ShapeDtypeStruct((n, d), jnp.float32),
    )(v, xc, yraw, ysc)


def _finish_call(acc, rest, rinv):
    n, d = rest.shape
    ca = ALPHA / (1.0 + ALPHA)

    def body(acc_ref, rest_ref, rinv_ref, out_ref):
        p1 = (acc_ref[0, :n, :] + acc_ref[1, :n, :]) * rinv_ref[...]
        out_ref[...] = (ca * (1.0 - BETA)) * p1 + rest_ref[...]

    return pl.pallas_call(
        body,
        out_shape=jax.ShapeDtypeStruct((n, d), jnp.float32),
    )(acc, rest, rinv)


def _final_call(v, weight, bias):
    n = v.shape[0]
    nout = weight.shape[1]

    def body(v_ref, w_ref, b_ref, out_ref):
        out_ref[...] = (jnp.dot(v_ref[...], w_ref[...],
                                preferred_element_type=jnp.float32)
                        + b_ref[...])

    return pl.pallas_call(
        body,
        out_shape=jax.ShapeDtypeStruct((n, nout), jnp.float32),
    )(v, weight, bias)


def kernel(x, edge_index, y, train_mask, weight, bias):
    n, d = x.shape
    e = edge_index.shape[1]
    et = e + n

    n_rows = -(-(n + 32) // (NS * CH)) * (NS * CH)
    per_tile = -(-et // NW)
    per_tile = -(-per_tile // G) * G
    nb = per_tile // G
    tot = per_tile * NW
    pad = tot - et

    sl = jnp.arange(n, dtype=jnp.int32)
    row = jnp.concatenate([edge_index[0], sl,
                           n + (jnp.arange(pad, dtype=jnp.int32) % 32)])
    col = jnp.concatenate([edge_index[1], sl,
                           jnp.arange(pad, dtype=jnp.int32) % n])

    zeros_blk = jnp.zeros((CH, d), dtype=jnp.float32)
    ones_tbl = jnp.ones((n, d), dtype=jnp.float32)
    iota = jnp.arange(n_rows, dtype=jnp.int32)

    deg2 = _spmm_call(ones_tbl, col, row, zeros_blk, iota, n_rows=n_rows, nb=nb)
    xc, rinv, yraw, ysc = _prep_call(
        x, deg2, y.reshape(n, 1),
        train_mask.reshape(n, 1).astype(jnp.float32))

    v = xc
    for _ in range(NPOW):
        acc = _spmm_call(v, col, row, zeros_blk, iota, n_rows=n_rows, nb=nb)
        rest = _rest_call(v, xc, yraw, ysc)
        v = _finish_call(acc, rest, rinv)

    return _final_call(v, weight, bias)


# speedup vs baseline: 10.9546x; 1.3420x over previous
import functools

import jax
import jax.numpy as jnp
from jax import lax
from jax.experimental import pallas as pl
from jax.experimental.pallas import tpu as pltpu
from jax.experimental.pallas import tpu_sc as plsc

NC = 2
NS = 16
NW = NC * NS
G = 128
CH = 128
NCLS = 40
ALPHA = 1.0
BETA = 1.0e-1
NPOW = 10


@functools.partial(jax.jit, static_argnames=("n_rows", "nb"))
def _spmm_call(v, colp, rowp, zeros_blk, iota, *, n_rows, nb):
    d = v.shape[1]
    nch = n_rows // CH
    mesh = plsc.VectorSubcoreMesh(core_axis_name="c", subcore_axis_name="s")

    @functools.partial(
        pl.kernel,
        out_type=jax.ShapeDtypeStruct((NC, n_rows, d), jnp.float32),
        mesh=mesh,
        scratch_types=[
            pltpu.VMEM((nb * G,), jnp.int32),
            pltpu.VMEM((nb, G), jnp.int32),
            pltpu.VMEM((G, d), jnp.float32),
            pltpu.VMEM_SHARED((n_rows, d), jnp.float32),
            pltpu.SemaphoreType.DMA,
            pltpu.SemaphoreType.DMA,
        ],
    )
    def spmm(v_hbm, colp_hbm, rowp_hbm, z_hbm, iota_hbm, acc_hbm,
             cidxf, ridx2, rows, acc_sh, sem, sem2):
        c = lax.axis_index("c")
        s = lax.axis_index("s")
        wid = s * NC + c

        rpt = n_rows // NS
        nck = rpt // CH
        pltpu.sync_copy(z_hbm, rows)
        for k in range(nck):
            pltpu.sync_copy(iota_hbm.at[pl.ds(s * rpt + k * CH, CH)], ridx2.at[0])
            pltpu.async_copy(rows, acc_sh.at[ridx2.at[0]], sem).wait()

        pltpu.sync_copy(colp_hbm.at[pl.ds(wid * nb * G, nb * G)], cidxf)
        pltpu.sync_copy(rowp_hbm.at[wid], ridx2)
        plsc.subcore_barrier()

        def body(b, carry):
            pltpu.async_copy(v_hbm.at[cidxf.at[pl.ds(b * G, G)]], rows, sem2).wait()
            pltpu.sync_copy(rows, acc_sh.at[ridx2.at[b]], add=True)
            return carry

        lax.fori_loop(0, nb, body, 0, unroll=False)
        plsc.subcore_barrier()

        for k in range(nck):
            pltpu.sync_copy(iota_hbm.at[pl.ds(s * rpt + k * CH, CH)], ridx2.at[0])
            pltpu.async_copy(acc_sh.at[ridx2.at[0]], rows, sem2).wait()
            pltpu.sync_copy(rows, acc_hbm.at[c, pl.ds(s * rpt + k * CH, CH)])

    return spmm(v, colp, rowp, zeros_blk, iota)


def _prep_call(x, deg2, y2, tm2):
    n, d = x.shape

    def body(x_ref, deg_ref, y_ref, tm_ref, xc_ref, rinv_ref, yraw_ref, ysc_ref):
        xv = x_ref[...]
        xc_ref[...] = xv - jnp.mean(xv, axis=0, keepdims=True)
        degv = deg_ref[0, :n, 0:1] + deg_ref[1, :n, 0:1]
        rinv_ref[...] = 1.0 / degv
        cls_ids = lax.broadcasted_iota(jnp.int32, (1, NCLS), 1)
        yv = jnp.where((y_ref[...] == cls_ids) & (tm_ref[...] > 0.5), 1.0, 0.0)
        yraw_ref[...] = yv
        cnt = jnp.sum(yv, axis=0, keepdims=True)
        ysc_ref[...] = yv / (cnt + 1e-8)

    return pl.pallas_call(
        body,
        out_shape=(
            jax.ShapeDtypeStruct((n, d), jnp.float32),
            jax.ShapeDtypeStruct((n, 1), jnp.float32),
            jax.ShapeDtypeStruct((n, NCLS), jnp.float32),
            jax.ShapeDtypeStruct((n, NCLS), jnp.float32),
        ),
    )(x, deg2, y2, tm2)


def _rest_call(v, xc, yraw, ysc):
    n, d = v.shape
    ca = ALPHA / (1.0 + ALPHA)
    cx = 1.0 / (1.0 + ALPHA)

    def body(v_ref, xc_ref, yraw_ref, ysc_ref, out_ref):
        vv = v_ref[...]
        cls = lax.dot_general(yraw_ref[...], vv, (((0,), (0,)), ((), ())),
                              preferred_element_type=jnp.float32)
        part2 = jnp.dot(ysc_ref[...], cls, preferred_element_type=jnp.float32)
        out_ref[...] = (ca * BETA) * part2 + cx * xc_ref[...]

    return pl.pallas_call(
        body,
        out_shape=jax.ShapeDtypeStruct((n, d), jnp.float32),
    )(v, xc, yraw, ysc)


def _finish_call(acc, rest, rinv):
    n, d = rest.shape
    ca = ALPHA / (1.0 + ALPHA)

    def body(acc_ref, rest_ref, rinv_ref, out_ref):
        p1 = (acc_ref[0, :n, :] + acc_ref[1, :n, :]) * rinv_ref[...]
        out_ref[...] = (ca * (1.0 - BETA)) * p1 + rest_ref[...]

    return pl.pallas_call(
        body,
        out_shape=jax.ShapeDtypeStruct((n, d), jnp.float32),
    )(acc, rest, rinv)


def _final_call(v, weight, bias):
    n = v.shape[0]
    nout = weight.shape[1]

    def body(v_ref, w_ref, b_ref, out_ref):
        out_ref[...] = (jnp.dot(v_ref[...], w_ref[...],
                                preferred_element_type=jnp.float32)
                        + b_ref[...])

    return pl.pallas_call(
        body,
        out_shape=jax.ShapeDtypeStruct((n, nout), jnp.float32),
    )(v, weight, bias)


def kernel(x, edge_index, y, train_mask, weight, bias):
    n, d = x.shape
    e = edge_index.shape[1]
    et = e + n

    n_rows = -(-(n + 32) // (NS * CH)) * (NS * CH)
    per_tile = -(-et // NW)
    per_tile = -(-per_tile // G) * G
    nb = per_tile // G
    tot = per_tile * NW
    pad = tot - et

    sl = jnp.arange(n, dtype=jnp.int32)
    row = jnp.concatenate([edge_index[0], sl,
                           n + (jnp.arange(pad, dtype=jnp.int32) % 32)])
    row = row.reshape(NW, nb, G)
    col = jnp.concatenate([edge_index[1], sl,
                           jnp.arange(pad, dtype=jnp.int32) % n])

    zeros_blk = jnp.zeros((CH, d), dtype=jnp.float32)
    ones_tbl = jnp.ones((n, d), dtype=jnp.float32)
    iota = jnp.arange(n_rows, dtype=jnp.int32)

    deg2 = _spmm_call(ones_tbl, col, row, zeros_blk, iota, n_rows=n_rows, nb=nb)
    xc, rinv, yraw, ysc = _prep_call(
        x, deg2, y.reshape(n, 1),
        train_mask.reshape(n, 1).astype(jnp.float32))

    v = xc
    for _ in range(NPOW):
        acc = _spmm_call(v, col, row, zeros_blk, iota, n_rows=n_rows, nb=nb)
        rest = _rest_call(v, xc, yraw, ysc)
        v = _finish_call(acc, rest, rinv)

    return _final_call(v, weight, bias)


# speedup vs baseline: 11.5801x; 1.0571x over previous
import functools

import jax
import jax.numpy as jnp
from jax import lax
from jax.experimental import pallas as pl
from jax.experimental.pallas import tpu as pltpu
from jax.experimental.pallas import tpu_sc as plsc

NC = 2
NS = 16
NW = NC * NS
G = 128
CH = 128
NCLS = 40
ALPHA = 1.0
BETA = 1.0e-1
NPOW = 10


@functools.partial(jax.jit, static_argnames=("n_rows", "nb"))
def _spmm_call(v, colp, rowp, zeros_blk, iota, *, n_rows, nb):
    d = v.shape[1]
    nch = n_rows // CH
    mesh = plsc.VectorSubcoreMesh(core_axis_name="c", subcore_axis_name="s")

    @functools.partial(
        pl.kernel,
        out_type=jax.ShapeDtypeStruct((NC, n_rows, d), jnp.float32),
        mesh=mesh,
        scratch_types=[
            pltpu.VMEM((nb * G,), jnp.int32),
            pltpu.VMEM((nb, G), jnp.int32),
            pltpu.VMEM((G, d), jnp.float32),
            pltpu.VMEM_SHARED((n_rows, d), jnp.float32),
            pltpu.SemaphoreType.DMA,
            pltpu.SemaphoreType.DMA,
        ],
    )
    def spmm(v_hbm, colp_hbm, rowp_hbm, z_hbm, iota_hbm, acc_hbm,
             cidxf, ridx2, rows, acc_sh, sem, sem2):
        c = lax.axis_index("c")
        s = lax.axis_index("s")
        wid = s * NC + c

        rpt = n_rows // NS
        nck = rpt // CH
        pltpu.sync_copy(z_hbm, rows)
        for k in range(nck):
            pltpu.sync_copy(iota_hbm.at[pl.ds(s * rpt + k * CH, CH)], ridx2.at[0])
            pltpu.async_copy(rows, acc_sh.at[ridx2.at[0]], sem).wait()

        pltpu.sync_copy(colp_hbm.at[pl.ds(wid * nb * G, nb * G)], cidxf)
        pltpu.sync_copy(rowp_hbm.at[wid], ridx2)
        plsc.subcore_barrier()

        def body(b, carry):
            pltpu.async_copy(v_hbm.at[cidxf.at[pl.ds(b * G, G)]], rows, sem2).wait()
            pltpu.sync_copy(rows, acc_sh.at[ridx2.at[b]], add=True)
            return carry

        lax.fori_loop(0, nb, body, 0, unroll=False)
        plsc.subcore_barrier()

        for k in range(nck):
            pltpu.sync_copy(iota_hbm.at[pl.ds(s * rpt + k * CH, CH)], ridx2.at[0])
            pltpu.async_copy(acc_sh.at[ridx2.at[0]], rows, sem2).wait()
            pltpu.sync_copy(rows, acc_hbm.at[c, pl.ds(s * rpt + k * CH, CH)])

    return spmm(v, colp, rowp, zeros_blk, iota)


@functools.partial(jax.jit, static_argnames=("n_rows", "nb"))
def _deg_call(rowp, ones_blk, zeros_blk, iota, *, n_rows, nb):
    d = ones_blk.shape[1]
    mesh = plsc.VectorSubcoreMesh(core_axis_name="c", subcore_axis_name="s")

    @functools.partial(
        pl.kernel,
        out_type=jax.ShapeDtypeStruct((NC, n_rows, d), jnp.float32),
        mesh=mesh,
        scratch_types=[
            pltpu.VMEM((nb, G), jnp.int32),
            pltpu.VMEM((G, d), jnp.float32),
            pltpu.VMEM_SHARED((n_rows, d), jnp.float32),
            pltpu.SemaphoreType.DMA,
            pltpu.SemaphoreType.DMA,
        ],
    )
    def deg(rowp_hbm, ones_hbm, z_hbm, iota_hbm, acc_hbm,
            ridx2, rows, acc_sh, sem, sem2):
        c = lax.axis_index("c")
        s = lax.axis_index("s")
        wid = s * NC + c

        rpt = n_rows // NS
        nck = rpt // CH
        pltpu.sync_copy(z_hbm, rows)
        for k in range(nck):
            pltpu.sync_copy(iota_hbm.at[pl.ds(s * rpt + k * CH, CH)], ridx2.at[0])
            pltpu.async_copy(rows, acc_sh.at[ridx2.at[0]], sem).wait()

        pltpu.sync_copy(rowp_hbm.at[wid], ridx2)
        pltpu.sync_copy(ones_hbm, rows)
        plsc.subcore_barrier()

        def body(b, carry):
            pltpu.sync_copy(rows, acc_sh.at[ridx2.at[b]], add=True)
            return carry

        lax.fori_loop(0, nb, body, 0, unroll=False)
        plsc.subcore_barrier()

        for k in range(nck):
            pltpu.sync_copy(iota_hbm.at[pl.ds(s * rpt + k * CH, CH)], ridx2.at[0])
            pltpu.async_copy(acc_sh.at[ridx2.at[0]], rows, sem2).wait()
            pltpu.sync_copy(rows, acc_hbm.at[c, pl.ds(s * rpt + k * CH, CH)])

    return deg(rowp, ones_blk, zeros_blk, iota)


def _prep_call(x, deg2, y2, tm2):
    n, d = x.shape

    def body(x_ref, deg_ref, y_ref, tm_ref, xc_ref, rinv_ref, yraw_ref, ysc_ref):
        xv = x_ref[...]
        xc_ref[...] = xv - jnp.mean(xv, axis=0, keepdims=True)
        degv = deg_ref[0, :n, 0:1] + deg_ref[1, :n, 0:1]
        rinv_ref[...] = 1.0 / degv
        cls_ids = lax.broadcasted_iota(jnp.int32, (1, NCLS), 1)
        yv = jnp.where((y_ref[...] == cls_ids) & (tm_ref[...] > 0.5), 1.0, 0.0)
        yraw_ref[...] = yv
        cnt = jnp.sum(yv, axis=0, keepdims=True)
        ysc_ref[...] = yv / (cnt + 1e-8)

    return pl.pallas_call(
        body,
        out_shape=(
            jax.ShapeDtypeStruct((n, d), jnp.float32),
            jax.ShapeDtypeStruct((n, 1), jnp.float32),
            jax.ShapeDtypeStruct((n, NCLS), jnp.float32),
            jax.ShapeDtypeStruct((n, NCLS), jnp.float32),
        ),
    )(x, deg2, y2, tm2)


def _rest_call(v, xc, yraw, ysc):
    n, d = v.shape
    ca = ALPHA / (1.0 + ALPHA)
    cx = 1.0 / (1.0 + ALPHA)

    def body(v_ref, xc_ref, yraw_ref, ysc_ref, out_ref):
        vv = v_ref[...]
        cls = lax.dot_general(yraw_ref[...], vv, (((0,), (0,)), ((), ())),
                              preferred_element_type=jnp.float32)
        part2 = jnp.dot(ysc_ref[...], cls, preferred_element_type=jnp.float32)
        out_ref[...] = (ca * BETA) * part2 + cx * xc_ref[...]

    return pl.pallas_call(
        body,
        out_shape=jax.ShapeDtypeStruct((n, d), jnp.float32),
    )(v, xc, yraw, ysc)


def _finish_call(acc, rest, rinv):
    n, d = rest.shape
    ca = ALPHA / (1.0 + ALPHA)

    def body(acc_ref, rest_ref, rinv_ref, out_ref):
        p1 = (acc_ref[0, :n, :] + acc_ref[1, :n, :]) * rinv_ref[...]
        out_ref[...] = (ca * (1.0 - BETA)) * p1 + rest_ref[...]

    return pl.pallas_call(
        body,
        out_shape=jax.ShapeDtypeStruct((n, d), jnp.float32),
    )(acc, rest, rinv)


def _final_call(v, weight, bias):
    n = v.shape[0]
    nout = weight.shape[1]

    def body(v_ref, w_ref, b_ref, out_ref):
        out_ref[...] = (jnp.dot(v_ref[...], w_ref[...],
                                preferred_element_type=jnp.float32)
                        + b_ref[...])

    return pl.pallas_call(
        body,
        out_shape=jax.ShapeDtypeStruct((n, nout), jnp.float32),
    )(v, weight, bias)


def kernel(x, edge_index, y, train_mask, weight, bias):
    n, d = x.shape
    e = edge_index.shape[1]
    et = e + n

    n_rows = -(-(n + 32) // (NS * CH)) * (NS * CH)
    per_tile = -(-et // NW)
    per_tile = -(-per_tile // G) * G
    nb = per_tile // G
    tot = per_tile * NW
    pad = tot - et

    sl = jnp.arange(n, dtype=jnp.int32)
    row = jnp.concatenate([edge_index[0], sl,
                           n + (jnp.arange(pad, dtype=jnp.int32) % 32)])
    row = row.reshape(NW, nb, G)
    col = jnp.concatenate([edge_index[1], sl,
                           jnp.arange(pad, dtype=jnp.int32) % n])

    zeros_blk = jnp.zeros((CH, d), dtype=jnp.float32)
    ones_blk = jnp.ones((G, d), dtype=jnp.float32)
    iota = jnp.arange(n_rows, dtype=jnp.int32)

    deg2 = _deg_call(row, ones_blk, zeros_blk, iota, n_rows=n_rows, nb=nb)
    xc, rinv, yraw, ysc = _prep_call(
        x, deg2, y.reshape(n, 1),
        train_mask.reshape(n, 1).astype(jnp.float32))

    v = xc
    for _ in range(NPOW):
        acc = _spmm_call(v, col, row, zeros_blk, iota, n_rows=n_rows, nb=nb)
        rest = _rest_call(v, xc, yraw, ysc)
        v = _finish_call(acc, rest, rinv)

    return _final_call(v, weight, bias)
